# initial kernel scaffold (unmeasured)
import jax
import jax.numpy as jnp
from jax import lax
from jax.experimental import pallas as pl
from jax.experimental.pallas import tpu as pltpu

N_DEV = 4


def kernel(x, w_mat):
    m_glob, k_loc = x.shape
    k2, n = w_mat.shape
    assert k_loc == k2
    m_blk = m_glob // N_DEV

    def body(x_ref, w_ref, out_ref, comm_ref, amax_ref,
             send_sems, recv_sems, amax_send_sems, amax_recv_sems):
        my = lax.axis_index("i")
        left = lax.rem(my + (N_DEV - 1), N_DEV)
        right = lax.rem(my + 1, N_DEV)

        barrier_sem = pltpu.get_barrier_semaphore()
        for nbr in (left, right):
            pl.semaphore_signal(
                barrier_sem, inc=1,
                device_id=(nbr,), device_id_type=pl.DeviceIdType.MESH,
            )
        pl.semaphore_wait(barrier_sem, 2)

        def partial(block_idx):
            xb = x_ref[pl.ds(block_idx * m_blk, m_blk), :]
            return lax.dot_general(
                xb, w_ref[...], (((1,), (0,)), ((), ())),
                preferred_element_type=jnp.float32,
                precision=lax.Precision.HIGHEST,
            )

        comm_ref[0] = partial(lax.rem(my + 3, N_DEV))

        for s in range(N_DEV - 1):
            rdma = pltpu.make_async_remote_copy(
                src_ref=comm_ref.at[s],
                dst_ref=comm_ref.at[s + 1],
                send_sem=send_sems.at[s],
                recv_sem=recv_sems.at[s],
                device_id=(right,),
                device_id_type=pl.DeviceIdType.MESH,
            )
            rdma.start()
            rdma.wait()
            p = partial(lax.rem(my + 2 - s, N_DEV))
            if s < N_DEV - 2:
                comm_ref[s + 1] = comm_ref[s + 1] + p
            else:
                out_ref[...] = jnp.maximum(comm_ref[s + 1] + p, 0.0)

        local_amax = jnp.max(out_ref[...])
        amax_ref[0] = jnp.full((8, 128), local_amax, jnp.float32)
        amax_rdmas = []
        for k in range(1, N_DEV):
            tgt = lax.rem(my + k, N_DEV)
            r = pltpu.make_async_remote_copy(
                src_ref=amax_ref.at[0],
                dst_ref=amax_ref.at[k],
                send_sem=amax_send_sems.at[k - 1],
                recv_sem=amax_recv_sems.at[k - 1],
                device_id=(tgt,),
                device_id_type=pl.DeviceIdType.MESH,
            )
            r.start()
            amax_rdmas.append(r)
        for r in amax_rdmas:
            r.wait()

        global_amax = jnp.max(amax_ref[:, 0, 0])
        scale = jnp.maximum(global_amax, 1e-30) / 448.0
        scaled = jnp.minimum(out_ref[...] / scale, 448.0)
        q = scaled.astype(jnp.float8_e4m3fn)
        out_ref[...] = q.astype(jnp.float32) * scale

    return pl.pallas_call(
        body,
        out_shape=jax.ShapeDtypeStruct((m_blk, n), jnp.float32),
        in_specs=[
            pl.BlockSpec(memory_space=pltpu.VMEM),
            pl.BlockSpec(memory_space=pltpu.VMEM),
        ],
        out_specs=pl.BlockSpec(memory_space=pltpu.VMEM),
        scratch_shapes=[
            pltpu.VMEM((N_DEV, m_blk, n), jnp.float32),
            pltpu.VMEM((N_DEV, 8, 128), jnp.float32),
            pltpu.SemaphoreType.DMA((N_DEV - 1,)),
            pltpu.SemaphoreType.DMA((N_DEV - 1,)),
            pltpu.SemaphoreType.DMA((N_DEV - 1,)),
            pltpu.SemaphoreType.DMA((N_DEV - 1,)),
        ],
        compiler_params=pltpu.CompilerParams(collective_id=0),
    )(x, w_mat)


# baseline (device time: 317318 ns/iter reference)
import jax
import jax.numpy as jnp
from jax import lax
from jax.experimental import pallas as pl
from jax.experimental.pallas import tpu as pltpu

N_DEV = 4
N_CHUNK = 512


def kernel(x, w_mat):
    m_glob, k_loc = x.shape
    k2, n = w_mat.shape
    assert k_loc == k2
    m_blk = m_glob // N_DEV
    n_chunks = n // N_CHUNK

    def body(x_hbm, w_ref, out_ref, stage_ref, comm_ref, amax_ref,
             x_sems, send_sems, recv_sems, amax_send_sems, amax_recv_sems,
             credit_sem):
        my = lax.axis_index("i")
        left = lax.rem(my + (N_DEV - 1), N_DEV)
        right = lax.rem(my + 1, N_DEV)

        barrier_sem = pltpu.get_barrier_semaphore()
        for nbr in (left, right):
            pl.semaphore_signal(
                barrier_sem, inc=1,
                device_id=(nbr,), device_id_type=pl.DeviceIdType.MESH,
            )
        pl.semaphore_wait(barrier_sem, 2)

        def fetch_x_block(block_idx, slot):
            return pltpu.make_async_copy(
                x_hbm.at[pl.ds(block_idx * m_blk, m_blk), :],
                stage_ref.at[slot],
                x_sems.at[slot],
            )

        def accum_block(slot, dst, init=False, relu=False):
            xv = stage_ref[slot]
            for c in range(n_chunks):
                cs = slice(c * N_CHUNK, (c + 1) * N_CHUNK)
                p = lax.dot_general(
                    xv, w_ref[:, cs], (((1,), (0,)), ((), ())),
                    preferred_element_type=jnp.float32,
                )
                if init:
                    dst[:, cs] = p
                elif relu:
                    dst[:, cs] = jnp.maximum(dst[:, cs] + p, 0.0)
                else:
                    dst[:, cs] = dst[:, cs] + p

        send_slot = (0, 1, 0)
        recv_slot = (1, 0, None)
        dma0 = fetch_x_block(lax.rem(my + 3, N_DEV), 0)
        dma0.start()
        dma0.wait()
        accum_block(0, comm_ref.at[0], init=True)

        for s in range(N_DEV - 1):
            last = s == N_DEV - 2
            if s == 1:
                pl.semaphore_wait(credit_sem, 1)
            rdma = pltpu.make_async_remote_copy(
                src_ref=comm_ref.at[send_slot[s]],
                dst_ref=out_ref if last else comm_ref.at[recv_slot[s]],
                send_sem=send_sems.at[s],
                recv_sem=recv_sems.at[s],
                device_id=(right,),
                device_id_type=pl.DeviceIdType.MESH,
            )
            rdma.start()
            dma = fetch_x_block(lax.rem(my + 2 - s, N_DEV), (s + 1) % 2)
            dma.start()
            rdma.wait()
            if s == 0:
                pl.semaphore_signal(
                    credit_sem, inc=1,
                    device_id=(left,), device_id_type=pl.DeviceIdType.MESH,
                )
            dma.wait()
            if last:
                accum_block((s + 1) % 2, out_ref, relu=True)
            else:
                accum_block((s + 1) % 2, comm_ref.at[recv_slot[s]])

        local_amax = jnp.float32(0.0)
        for c in range(n_chunks):
            cs = slice(c * N_CHUNK, (c + 1) * N_CHUNK)
            local_amax = jnp.maximum(local_amax, jnp.max(out_ref[:, cs]))
        amax_ref[0] = jnp.full((8, 128), local_amax, jnp.float32)
        amax_rdmas = []
        for k in range(1, N_DEV):
            tgt = lax.rem(my + k, N_DEV)
            r = pltpu.make_async_remote_copy(
                src_ref=amax_ref.at[0],
                dst_ref=amax_ref.at[k],
                send_sem=amax_send_sems.at[k - 1],
                recv_sem=amax_recv_sems.at[k - 1],
                device_id=(tgt,),
                device_id_type=pl.DeviceIdType.MESH,
            )
            r.start()
            amax_rdmas.append(r)
        for r in amax_rdmas:
            r.wait()

        global_amax = jnp.max(amax_ref[:, 0, 0])
        scale = jnp.maximum(global_amax, 1e-30) / 448.0
        for c in range(n_chunks):
            cs = slice(c * N_CHUNK, (c + 1) * N_CHUNK)
            scaled = jnp.minimum(out_ref[:, cs] / scale, 448.0)
            q = scaled.astype(jnp.float8_e4m3fn)
            out_ref[:, cs] = q.astype(jnp.float32) * scale

    return pl.pallas_call(
        body,
        out_shape=jax.ShapeDtypeStruct((m_blk, n), jnp.float32),
        in_specs=[
            pl.BlockSpec(memory_space=pltpu.MemorySpace.HBM),
            pl.BlockSpec(memory_space=pltpu.VMEM),
        ],
        out_specs=pl.BlockSpec(memory_space=pltpu.VMEM),
        scratch_shapes=[
            pltpu.VMEM((2, m_blk, k_loc), jnp.float32),
            pltpu.VMEM((2, m_blk, n), jnp.float32),
            pltpu.VMEM((N_DEV, 8, 128), jnp.float32),
            pltpu.SemaphoreType.DMA((2,)),
            pltpu.SemaphoreType.DMA((N_DEV - 1,)),
            pltpu.SemaphoreType.DMA((N_DEV - 1,)),
            pltpu.SemaphoreType.DMA((N_DEV - 1,)),
            pltpu.SemaphoreType.DMA((N_DEV - 1,)),
            pltpu.SemaphoreType.REGULAR,
        ],
        compiler_params=pltpu.CompilerParams(
            collective_id=0,
            vmem_limit_bytes=46 * 1024 * 1024,
        ),
    )(x, w_mat)


# device time: 183424 ns/iter; 1.7300x vs baseline; 1.7300x over previous
import jax
import jax.numpy as jnp
from jax import lax
from jax.experimental import pallas as pl
from jax.experimental.pallas import tpu as pltpu

N_DEV = 4
N_CHUNK = 512


def kernel(x, w_mat):
    m_glob, k_loc = x.shape
    k2, n = w_mat.shape
    assert k_loc == k2
    m_blk = m_glob // N_DEV
    n_half = n // 2

    def body(x_hbm, w_ref, out_ref, stage_ref, cw_ref, ccw_ref, amax_ref,
             x_sems, cw_send_sems, cw_recv_sems, ccw_send_sems, ccw_recv_sems,
             amax_send_sems, amax_recv_sems, credit_cw, credit_ccw):
        my = lax.axis_index("i")
        left = lax.rem(my + (N_DEV - 1), N_DEV)
        right = lax.rem(my + 1, N_DEV)

        barrier_sem = pltpu.get_barrier_semaphore()
        for nbr in (left, right):
            pl.semaphore_signal(
                barrier_sem, inc=1,
                device_id=(nbr,), device_id_type=pl.DeviceIdType.MESH,
            )
        pl.semaphore_wait(barrier_sem, 2)

        def fetch_x_block(block_idx, slot):
            return pltpu.make_async_copy(
                x_hbm.at[pl.ds(block_idx * m_blk, m_blk), :],
                stage_ref.at[slot],
                x_sems.at[slot],
            )

        def accum_half(stage_slot, dst, w_col0, init=False, relu=False):
            xv = stage_ref[stage_slot]
            for c in range(n_half // N_CHUNK):
                ds_ = slice(c * N_CHUNK, (c + 1) * N_CHUNK)
                ws_ = slice(w_col0 + c * N_CHUNK, w_col0 + (c + 1) * N_CHUNK)
                p = lax.dot_general(
                    xv, w_ref[:, ws_], (((1,), (0,)), ((), ())),
                    preferred_element_type=jnp.float32,
                )
                if init:
                    dst[:, ds_] = p
                elif relu:
                    dst[:, ds_] = jnp.maximum(dst[:, ds_] + p, 0.0)
                else:
                    dst[:, ds_] = dst[:, ds_] + p

        send_slot = (0, 1, 0)
        recv_slot = (1, 0, None)

        dmas = [fetch_x_block(lax.rem(my + 3, N_DEV), 0),
                fetch_x_block(lax.rem(my + 1, N_DEV), 1),
                fetch_x_block(lax.rem(my + 2, N_DEV), 2)]
        for d in dmas:
            d.start()
        dmas[0].wait()
        accum_half(0, cw_ref.at[0], 0, init=True)
        dmas[1].wait()
        accum_half(1, ccw_ref.at[0], n_half, init=True)

        cw_stage = (2, 1, 2)
        ccw_stage = (2, 0, 2)

        for s in range(N_DEV - 1):
            last = s == N_DEV - 2
            if s == 1:
                pl.semaphore_wait(credit_cw, 1)
                pl.semaphore_wait(credit_ccw, 1)
            cw_rdma = pltpu.make_async_remote_copy(
                src_ref=cw_ref.at[send_slot[s]],
                dst_ref=(out_ref.at[:, pl.ds(0, n_half)] if last
                         else cw_ref.at[recv_slot[s]]),
                send_sem=cw_send_sems.at[s],
                recv_sem=cw_recv_sems.at[s],
                device_id=(right,),
                device_id_type=pl.DeviceIdType.MESH,
            )
            cw_rdma.start()
            ccw_rdma = pltpu.make_async_remote_copy(
                src_ref=ccw_ref.at[send_slot[s]],
                dst_ref=(out_ref.at[:, pl.ds(n_half, n_half)] if last
                         else ccw_ref.at[recv_slot[s]]),
                send_sem=ccw_send_sems.at[s],
                recv_sem=ccw_recv_sems.at[s],
                device_id=(left,),
                device_id_type=pl.DeviceIdType.MESH,
            )
            ccw_rdma.start()
            if s == 0:
                dmas[2].wait()
            if last:
                dma_my.wait()

            cw_rdma.wait()
            if s == 0:
                pl.semaphore_signal(
                    credit_cw, inc=1,
                    device_id=(left,), device_id_type=pl.DeviceIdType.MESH,
                )
            accum_half(
                cw_stage[s],
                out_ref.at[:, pl.ds(0, n_half)] if last
                else cw_ref.at[recv_slot[s]],
                0, relu=last,
            )
            ccw_rdma.wait()
            if s == 0:
                pl.semaphore_signal(
                    credit_ccw, inc=1,
                    device_id=(right,), device_id_type=pl.DeviceIdType.MESH,
                )
            accum_half(
                ccw_stage[s],
                out_ref.at[:, pl.ds(n_half, n_half)] if last
                else ccw_ref.at[recv_slot[s]],
                n_half, relu=last,
            )
            if s == 0:
                dma_my = fetch_x_block(my, 2)
                dma_my.start()

        local_amax = jnp.float32(0.0)
        for c in range(n // N_CHUNK):
            cs = slice(c * N_CHUNK, (c + 1) * N_CHUNK)
            local_amax = jnp.maximum(local_amax, jnp.max(out_ref[:, cs]))
        amax_ref[0] = jnp.full((8, 128), local_amax, jnp.float32)
        amax_rdmas = []
        for k in range(1, N_DEV):
            tgt = lax.rem(my + k, N_DEV)
            r = pltpu.make_async_remote_copy(
                src_ref=amax_ref.at[0],
                dst_ref=amax_ref.at[k],
                send_sem=amax_send_sems.at[k - 1],
                recv_sem=amax_recv_sems.at[k - 1],
                device_id=(tgt,),
                device_id_type=pl.DeviceIdType.MESH,
            )
            r.start()
            amax_rdmas.append(r)
        for r in amax_rdmas:
            r.wait()

        global_amax = jnp.max(amax_ref[:, 0, 0])
        scale = jnp.maximum(global_amax, 1e-30) / 448.0
        for c in range(n // N_CHUNK):
            cs = slice(c * N_CHUNK, (c + 1) * N_CHUNK)
            scaled = jnp.minimum(out_ref[:, cs] / scale, 448.0)
            q = scaled.astype(jnp.float8_e4m3fn)
            out_ref[:, cs] = q.astype(jnp.float32) * scale

    return pl.pallas_call(
        body,
        out_shape=jax.ShapeDtypeStruct((m_blk, n), jnp.float32),
        in_specs=[
            pl.BlockSpec(memory_space=pltpu.MemorySpace.HBM),
            pl.BlockSpec(memory_space=pltpu.VMEM),
        ],
        out_specs=pl.BlockSpec(memory_space=pltpu.VMEM),
        scratch_shapes=[
            pltpu.VMEM((3, m_blk, k_loc), jnp.float32),
            pltpu.VMEM((2, m_blk, n_half), jnp.float32),
            pltpu.VMEM((2, m_blk, n_half), jnp.float32),
            pltpu.VMEM((N_DEV, 8, 128), jnp.float32),
            pltpu.SemaphoreType.DMA((3,)),
            pltpu.SemaphoreType.DMA((N_DEV - 1,)),
            pltpu.SemaphoreType.DMA((N_DEV - 1,)),
            pltpu.SemaphoreType.DMA((N_DEV - 1,)),
            pltpu.SemaphoreType.DMA((N_DEV - 1,)),
            pltpu.SemaphoreType.DMA((N_DEV - 1,)),
            pltpu.SemaphoreType.DMA((N_DEV - 1,)),
            pltpu.SemaphoreType.REGULAR,
            pltpu.SemaphoreType.REGULAR,
        ],
        compiler_params=pltpu.CompilerParams(
            collective_id=0,
            vmem_limit_bytes=47 * 1024 * 1024,
        ),
    )(x, w_mat)


# device time: 182665 ns/iter; 1.7372x vs baseline; 1.0042x over previous
import jax
import jax.numpy as jnp
from jax import lax
from jax.experimental import pallas as pl
from jax.experimental.pallas import tpu as pltpu

N_DEV = 4
N_CHUNK = 512


def kernel(x, w_mat):
    m_glob, k_loc = x.shape
    k2, n = w_mat.shape
    assert k_loc == k2
    m_blk = m_glob // N_DEV
    n_half = n // 2

    x = x.astype(jnp.bfloat16)
    w_mat = w_mat.astype(jnp.bfloat16)

    def body(x_hbm, w_ref, out_ref, stage_ref, cw_ref, ccw_ref, amax_ref,
             x_sems, cw_send_sems, cw_recv_sems, ccw_send_sems, ccw_recv_sems,
             amax_send_sems, amax_recv_sems, credit_cw, credit_ccw):
        my = lax.axis_index("i")
        left = lax.rem(my + (N_DEV - 1), N_DEV)
        right = lax.rem(my + 1, N_DEV)

        barrier_sem = pltpu.get_barrier_semaphore()
        for nbr in (left, right):
            pl.semaphore_signal(
                barrier_sem, inc=1,
                device_id=(nbr,), device_id_type=pl.DeviceIdType.MESH,
            )
        pl.semaphore_wait(barrier_sem, 2)

        def fetch_x_block(block_idx, slot):
            return pltpu.make_async_copy(
                x_hbm.at[pl.ds(block_idx * m_blk, m_blk), :],
                stage_ref.at[slot],
                x_sems.at[slot],
            )

        def accum_half(stage_slot, dst, w_col0, init=False, relu=False):
            xv = stage_ref[stage_slot]
            for c in range(n_half // N_CHUNK):
                ds_ = slice(c * N_CHUNK, (c + 1) * N_CHUNK)
                ws_ = slice(w_col0 + c * N_CHUNK, w_col0 + (c + 1) * N_CHUNK)
                p = lax.dot_general(
                    xv, w_ref[:, ws_], (((1,), (0,)), ((), ())),
                    preferred_element_type=jnp.float32,
                )
                if init:
                    dst[:, ds_] = p
                elif relu:
                    dst[:, ds_] = jnp.maximum(dst[:, ds_] + p, 0.0)
                else:
                    dst[:, ds_] = dst[:, ds_] + p

        send_slot = (0, 1, 0)
        recv_slot = (1, 0, None)

        def make_cw(s):
            last = s == N_DEV - 2
            return pltpu.make_async_remote_copy(
                src_ref=cw_ref.at[send_slot[s]],
                dst_ref=(out_ref.at[:, pl.ds(0, n_half)] if last
                         else cw_ref.at[recv_slot[s]]),
                send_sem=cw_send_sems.at[s],
                recv_sem=cw_recv_sems.at[s],
                device_id=(right,),
                device_id_type=pl.DeviceIdType.MESH,
            )

        def make_ccw(s):
            last = s == N_DEV - 2
            return pltpu.make_async_remote_copy(
                src_ref=ccw_ref.at[send_slot[s]],
                dst_ref=(out_ref.at[:, pl.ds(n_half, n_half)] if last
                         else ccw_ref.at[recv_slot[s]]),
                send_sem=ccw_send_sems.at[s],
                recv_sem=ccw_recv_sems.at[s],
                device_id=(left,),
                device_id_type=pl.DeviceIdType.MESH,
            )

        def partial_chunks(stage_slot, w_col0):
            xv = stage_ref[stage_slot]
            out = []
            for c in range(n_half // N_CHUNK):
                ws_ = slice(w_col0 + c * N_CHUNK, w_col0 + (c + 1) * N_CHUNK)
                out.append(lax.dot_general(
                    xv, w_ref[:, ws_], (((1,), (0,)), ((), ())),
                    preferred_element_type=jnp.float32,
                ))
            return out

        def apply_chunks(ps, dst, relu=False):
            for c, p in enumerate(ps):
                ds_ = slice(c * N_CHUNK, (c + 1) * N_CHUNK)
                if relu:
                    dst[:, ds_] = jnp.maximum(dst[:, ds_] + p, 0.0)
                else:
                    dst[:, ds_] = dst[:, ds_] + p

        dmas = [fetch_x_block(lax.rem(my + 3, N_DEV), 0),
                fetch_x_block(lax.rem(my + 1, N_DEV), 1),
                fetch_x_block(lax.rem(my + 2, N_DEV), 2)]
        for d in dmas:
            d.start()
        dmas[0].wait()
        accum_half(0, cw_ref.at[0], 0, init=True)
        cw_r = make_cw(0)
        cw_r.start()
        dmas[1].wait()
        accum_half(1, ccw_ref.at[0], n_half, init=True)
        ccw_r = make_ccw(0)
        ccw_r.start()

        cw_stage = (2, 1, 2)
        ccw_stage = (2, 0, 2)

        local_amax = jnp.float32(0.0)
        for s in range(N_DEV - 1):
            last = s == N_DEV - 2
            if s == 0:
                dmas[2].wait()
            if last:
                dma_my.wait()
            pcw = partial_chunks(cw_stage[s], 0)
            pccw = partial_chunks(ccw_stage[s], n_half)

            cw_r.wait()
            if s == 0:
                pl.semaphore_signal(
                    credit_cw, inc=1,
                    device_id=(left,), device_id_type=pl.DeviceIdType.MESH,
                )
            cw_dst = (out_ref.at[:, pl.ds(0, n_half)] if last
                      else cw_ref.at[recv_slot[s]])
            apply_chunks(pcw, cw_dst, relu=last)
            if last:
                for c in range(n_half // N_CHUNK):
                    cs = slice(c * N_CHUNK, (c + 1) * N_CHUNK)
                    local_amax = jnp.maximum(
                        local_amax, jnp.max(out_ref[:, cs]))
            else:
                if s == 0:
                    pl.semaphore_wait(credit_cw, 1)
                cw_r = make_cw(s + 1)
                cw_r.start()

            ccw_r.wait()
            if s == 0:
                pl.semaphore_signal(
                    credit_ccw, inc=1,
                    device_id=(right,), device_id_type=pl.DeviceIdType.MESH,
                )
            ccw_dst = (out_ref.at[:, pl.ds(n_half, n_half)] if last
                       else ccw_ref.at[recv_slot[s]])
            apply_chunks(pccw, ccw_dst, relu=last)
            if not last:
                if s == 0:
                    pl.semaphore_wait(credit_ccw, 1)
                ccw_r = make_ccw(s + 1)
                ccw_r.start()
            if s == 0:
                dma_my = fetch_x_block(my, 2)
                dma_my.start()

        for c in range(n_half // N_CHUNK):
            cs = slice(n_half + c * N_CHUNK, n_half + (c + 1) * N_CHUNK)
            local_amax = jnp.maximum(local_amax, jnp.max(out_ref[:, cs]))
        amax_ref[0] = jnp.full((8, 128), local_amax, jnp.float32)
        amax_rdmas = []
        for k in range(1, N_DEV):
            tgt = lax.rem(my + k, N_DEV)
            r = pltpu.make_async_remote_copy(
                src_ref=amax_ref.at[0],
                dst_ref=amax_ref.at[k],
                send_sem=amax_send_sems.at[k - 1],
                recv_sem=amax_recv_sems.at[k - 1],
                device_id=(tgt,),
                device_id_type=pl.DeviceIdType.MESH,
            )
            r.start()
            amax_rdmas.append(r)
        for r in amax_rdmas:
            r.wait()

        global_amax = jnp.max(amax_ref[:, 0, 0])
        scale = jnp.maximum(global_amax, 1e-30) / 448.0
        for c in range(n // N_CHUNK):
            cs = slice(c * N_CHUNK, (c + 1) * N_CHUNK)
            scaled = jnp.minimum(out_ref[:, cs] / scale, 448.0)
            q = scaled.astype(jnp.float8_e4m3fn)
            out_ref[:, cs] = q.astype(jnp.float32) * scale

    return pl.pallas_call(
        body,
        out_shape=jax.ShapeDtypeStruct((m_blk, n), jnp.float32),
        in_specs=[
            pl.BlockSpec(memory_space=pltpu.MemorySpace.HBM),
            pl.BlockSpec(memory_space=pltpu.VMEM),
        ],
        out_specs=pl.BlockSpec(memory_space=pltpu.VMEM),
        scratch_shapes=[
            pltpu.VMEM((3, m_blk, k_loc), jnp.bfloat16),
            pltpu.VMEM((2, m_blk, n_half), jnp.float32),
            pltpu.VMEM((2, m_blk, n_half), jnp.float32),
            pltpu.VMEM((N_DEV, 8, 128), jnp.float32),
            pltpu.SemaphoreType.DMA((3,)),
            pltpu.SemaphoreType.DMA((N_DEV - 1,)),
            pltpu.SemaphoreType.DMA((N_DEV - 1,)),
            pltpu.SemaphoreType.DMA((N_DEV - 1,)),
            pltpu.SemaphoreType.DMA((N_DEV - 1,)),
            pltpu.SemaphoreType.DMA((N_DEV - 1,)),
            pltpu.SemaphoreType.DMA((N_DEV - 1,)),
            pltpu.SemaphoreType.REGULAR,
            pltpu.SemaphoreType.REGULAR,
        ],
        compiler_params=pltpu.CompilerParams(
            collective_id=0,
            vmem_limit_bytes=51 * 1024 * 1024,
        ),
    )(x, w_mat)


# device time: 177270 ns/iter; 1.7900x vs baseline; 1.0304x over previous
import jax
import jax.numpy as jnp
from jax import lax
from jax.experimental import pallas as pl
from jax.experimental.pallas import tpu as pltpu

N_DEV = 4
N_CHUNK = 512
CH = 2


def kernel(x, w_mat):
    m_glob, k_loc = x.shape
    k2, n = w_mat.shape
    assert k_loc == k2
    m_blk = m_glob // N_DEV
    n_half = n // 2
    assert n_half == CH * N_CHUNK

    x = x.astype(jnp.bfloat16)
    w_mat = w_mat.astype(jnp.bfloat16)

    def body(x_hbm, w_ref, out_ref, stage_ref, cw_ref, ccw_ref, amax_ref,
             x_sems, cw_send_sems, cw_recv_sems, ccw_send_sems, ccw_recv_sems,
             amax_send_sems, amax_recv_sems):
        my = lax.axis_index("i")
        left = lax.rem(my + (N_DEV - 1), N_DEV)
        right = lax.rem(my + 1, N_DEV)

        barrier_sem = pltpu.get_barrier_semaphore()
        for nbr in (left, right):
            pl.semaphore_signal(
                barrier_sem, inc=1,
                device_id=(nbr,), device_id_type=pl.DeviceIdType.MESH,
            )
        pl.semaphore_wait(barrier_sem, 2)

        def fetch_x_block(block_idx, slot):
            return pltpu.make_async_copy(
                x_hbm.at[pl.ds(block_idx * m_blk, m_blk), :],
                stage_ref.at[slot],
                x_sems.at[slot],
            )

        def chunk_dot(stage_slot, w_col0, c):
            ws_ = slice(w_col0 + c * N_CHUNK, w_col0 + (c + 1) * N_CHUNK)
            return lax.dot_general(
                stage_ref[stage_slot], w_ref[:, ws_], (((1,), (0,)), ((), ())),
                preferred_element_type=jnp.float32,
            )

        def make_hop(lane, s, c):
            ref = cw_ref if lane == 0 else ccw_ref
            ssem = cw_send_sems if lane == 0 else ccw_send_sems
            rsem = cw_recv_sems if lane == 0 else ccw_recv_sems
            tgt = right if lane == 0 else left
            base = 0 if lane == 0 else n_half
            last = s == N_DEV - 2
            return pltpu.make_async_remote_copy(
                src_ref=ref.at[s, c],
                dst_ref=(out_ref.at[:, pl.ds(base + c * N_CHUNK, N_CHUNK)]
                         if last else ref.at[s + 1, c]),
                send_sem=ssem.at[CH * s + c],
                recv_sem=rsem.at[CH * s + c],
                device_id=(tgt,),
                device_id_type=pl.DeviceIdType.MESH,
            )

        dmas = [fetch_x_block(lax.rem(my + 3, N_DEV), 0),
                fetch_x_block(lax.rem(my + 1, N_DEV), 1),
                fetch_x_block(lax.rem(my + 2, N_DEV), 2)]
        for d in dmas:
            d.start()

        rdmas = {}
        dmas[0].wait()
        for c in range(CH):
            cw_ref[0, c] = chunk_dot(0, 0, c)
            r = make_hop(0, 0, c)
            r.start()
            rdmas[(0, 0, c)] = r
        dmas[1].wait()
        for c in range(CH):
            ccw_ref[0, c] = chunk_dot(1, n_half, c)
            r = make_hop(1, 0, c)
            r.start()
            rdmas[(1, 0, c)] = r

        cw_stage = (2, 1, 2)
        ccw_stage = (2, 0, 2)

        local_amax = jnp.float32(0.0)
        for s in range(N_DEV - 1):
            last = s == N_DEV - 2
            if s == 0:
                dmas[2].wait()
            if last:
                dma_my.wait()
            for c in range(CH):
                for lane in range(2):
                    ref = cw_ref if lane == 0 else ccw_ref
                    st = (cw_stage if lane == 0 else ccw_stage)[s]
                    base = 0 if lane == 0 else n_half
                    cs = slice(base + c * N_CHUNK, base + (c + 1) * N_CHUNK)
                    p = chunk_dot(st, base, c)
                    rdmas[(lane, s, c)].wait()
                    if last:
                        v = jnp.maximum(out_ref[:, cs] + p, 0.0)
                        out_ref[:, cs] = v
                        local_amax = jnp.maximum(local_amax, jnp.max(v))
                    else:
                        ref[s + 1, c] = ref[s + 1, c] + p
                        nr = make_hop(lane, s + 1, c)
                        nr.start()
                        rdmas[(lane, s + 1, c)] = nr
            if s == 0:
                dma_my = fetch_x_block(my, 2)
                dma_my.start()

        amax_ref[0] = jnp.full((8, 128), local_amax, jnp.float32)
        amax_rdmas = []
        for k in range(1, N_DEV):
            tgt = lax.rem(my + k, N_DEV)
            r = pltpu.make_async_remote_copy(
                src_ref=amax_ref.at[0],
                dst_ref=amax_ref.at[k],
                send_sem=amax_send_sems.at[k - 1],
                recv_sem=amax_recv_sems.at[k - 1],
                device_id=(tgt,),
                device_id_type=pl.DeviceIdType.MESH,
            )
            r.start()
            amax_rdmas.append(r)
        for r in amax_rdmas:
            r.wait()

        global_amax = jnp.max(amax_ref[:, 0, 0])
        scale = jnp.maximum(global_amax, 1e-30) / 448.0
        for c in range(n // N_CHUNK):
            cs = slice(c * N_CHUNK, (c + 1) * N_CHUNK)
            scaled = jnp.minimum(out_ref[:, cs] / scale, 448.0)
            q = scaled.astype(jnp.float8_e4m3fn)
            out_ref[:, cs] = q.astype(jnp.float32) * scale

    return pl.pallas_call(
        body,
        out_shape=jax.ShapeDtypeStruct((m_blk, n), jnp.float32),
        in_specs=[
            pl.BlockSpec(memory_space=pltpu.MemorySpace.HBM),
            pl.BlockSpec(memory_space=pltpu.VMEM),
        ],
        out_specs=pl.BlockSpec(memory_space=pltpu.VMEM),
        scratch_shapes=[
            pltpu.VMEM((3, m_blk, k_loc), jnp.bfloat16),
            pltpu.VMEM((N_DEV - 1, CH, m_blk, N_CHUNK), jnp.float32),
            pltpu.VMEM((N_DEV - 1, CH, m_blk, N_CHUNK), jnp.float32),
            pltpu.VMEM((N_DEV, 8, 128), jnp.float32),
            pltpu.SemaphoreType.DMA((3,)),
            pltpu.SemaphoreType.DMA(((N_DEV - 1) * CH,)),
            pltpu.SemaphoreType.DMA(((N_DEV - 1) * CH,)),
            pltpu.SemaphoreType.DMA(((N_DEV - 1) * CH,)),
            pltpu.SemaphoreType.DMA(((N_DEV - 1) * CH,)),
            pltpu.SemaphoreType.DMA((N_DEV - 1,)),
            pltpu.SemaphoreType.DMA((N_DEV - 1,)),
        ],
        compiler_params=pltpu.CompilerParams(
            collective_id=0,
            vmem_limit_bytes=51 * 1024 * 1024,
        ),
    )(x, w_mat)


# device time: 172481 ns/iter; 1.8397x vs baseline; 1.0278x over previous
import jax
import jax.numpy as jnp
from jax import lax
from jax.experimental import pallas as pl
from jax.experimental.pallas import tpu as pltpu

N_DEV = 4
N_CHUNK = 512
CH = 2


def kernel(x, w_mat):
    m_glob, k_loc = x.shape
    k2, n = w_mat.shape
    assert k_loc == k2
    m_blk = m_glob // N_DEV
    n_half = n // 2
    assert n_half == CH * N_CHUNK

    w_mat = w_mat.astype(jnp.bfloat16)

    def body(x_hbm, wb_ref, out_ref, stage_ref, xb_ref,
             cw_ref, ccw_ref, amax_ref,
             x_sems, cw_send_sems, cw_recv_sems, ccw_send_sems, ccw_recv_sems,
             amax_send_sems, amax_recv_sems, credit_cw, credit_ccw):
        my = lax.axis_index("i")
        left = lax.rem(my + (N_DEV - 1), N_DEV)
        right = lax.rem(my + 1, N_DEV)

        barrier_sem = pltpu.get_barrier_semaphore()
        for nbr in (left, right):
            pl.semaphore_signal(
                barrier_sem, inc=1,
                device_id=(nbr,), device_id_type=pl.DeviceIdType.MESH,
            )
        pl.semaphore_wait(barrier_sem, 2)

        def fetch_x_block(block_idx, slot):
            return pltpu.make_async_copy(
                x_hbm.at[pl.ds(block_idx * m_blk, m_blk), :],
                stage_ref.at[slot],
                x_sems.at[slot],
            )

        def cast_block(stage_slot, xb_slot):
            for r in range(4):
                rs = slice(r * (m_blk // 4), (r + 1) * (m_blk // 4))
                xb_ref[xb_slot, rs] = stage_ref[stage_slot, rs].astype(
                    jnp.bfloat16)

        def chunk_dot(slot, w_col0, c):
            ws_ = slice(w_col0 + c * N_CHUNK, w_col0 + (c + 1) * N_CHUNK)
            return lax.dot_general(
                xb_ref[slot], wb_ref[:, ws_], (((1,), (0,)), ((), ())),
                preferred_element_type=jnp.float32,
            )

        send_slot = (0, 1, 0)
        recv_slot = (1, 0, None)

        def make_hop(lane, s, c):
            ref = cw_ref if lane == 0 else ccw_ref
            ssem = cw_send_sems if lane == 0 else ccw_send_sems
            rsem = cw_recv_sems if lane == 0 else ccw_recv_sems
            tgt = right if lane == 0 else left
            base = 0 if lane == 0 else n_half
            last = s == N_DEV - 2
            return pltpu.make_async_remote_copy(
                src_ref=ref.at[send_slot[s], c],
                dst_ref=(out_ref.at[:, pl.ds(base + c * N_CHUNK, N_CHUNK)]
                         if last else ref.at[recv_slot[s], c]),
                send_sem=ssem.at[CH * s + c],
                recv_sem=rsem.at[CH * s + c],
                device_id=(tgt,),
                device_id_type=pl.DeviceIdType.MESH,
            )

        dma_a = fetch_x_block(lax.rem(my + 3, N_DEV), 0)
        dma_b = fetch_x_block(lax.rem(my + 1, N_DEV), 1)
        dma_a.start()
        dma_b.start()

        rdmas = {}
        dma_a.wait()
        cast_block(0, 0)
        for c in range(CH):
            cw_ref[0, c] = chunk_dot(0, 0, c)
            r = make_hop(0, 0, c)
            r.start()
            rdmas[(0, 0, c)] = r
        dma_c = fetch_x_block(lax.rem(my + 2, N_DEV), 0)
        dma_c.start()
        dma_b.wait()
        cast_block(1, 1)
        for c in range(CH):
            ccw_ref[0, c] = chunk_dot(1, n_half, c)
            r = make_hop(1, 0, c)
            r.start()
            rdmas[(1, 0, c)] = r
        dma_my = fetch_x_block(my, 1)
        dma_my.start()

        cw_stage = (2, 1, 2)
        ccw_stage = (2, 0, 2)

        local_amax = jnp.float32(0.0)
        for s in range(N_DEV - 1):
            last = s == N_DEV - 2
            if s == 0:
                dma_c.wait()
                cast_block(0, 2)
            if last:
                dma_my.wait()
                cast_block(1, 2)
            for c in range(CH):
                for lane in range(2):
                    ref = cw_ref if lane == 0 else ccw_ref
                    st = (cw_stage if lane == 0 else ccw_stage)[s]
                    base = 0 if lane == 0 else n_half
                    cs = slice(base + c * N_CHUNK, base + (c + 1) * N_CHUNK)
                    p = chunk_dot(st, base, c)
                    rdmas[(lane, s, c)].wait()
                    if last:
                        v = jnp.maximum(out_ref[:, cs] + p, 0.0)
                        out_ref[:, cs] = v
                        local_amax = jnp.maximum(local_amax, jnp.max(v))
                    else:
                        ref[recv_slot[s], c] = ref[recv_slot[s], c] + p
                        if s == 1:
                            nr = make_hop(lane, 2, c)
                            nr.start()
                            rdmas[(lane, 2, c)] = nr
            if s == 0:
                pl.semaphore_signal(
                    credit_cw, inc=1,
                    device_id=(left,), device_id_type=pl.DeviceIdType.MESH,
                )
                pl.semaphore_signal(
                    credit_ccw, inc=1,
                    device_id=(right,), device_id_type=pl.DeviceIdType.MESH,
                )
                pl.semaphore_wait(credit_cw, 1)
                pl.semaphore_wait(credit_ccw, 1)
                for c in range(CH):
                    for lane in range(2):
                        nr = make_hop(lane, 1, c)
                        nr.start()
                        rdmas[(lane, 1, c)] = nr

        amax_ref[0] = jnp.full((8, 128), local_amax, jnp.float32)
        amax_rdmas = []
        for k in range(1, N_DEV):
            tgt = lax.rem(my + k, N_DEV)
            r = pltpu.make_async_remote_copy(
                src_ref=amax_ref.at[0],
                dst_ref=amax_ref.at[k],
                send_sem=amax_send_sems.at[k - 1],
                recv_sem=amax_recv_sems.at[k - 1],
                device_id=(tgt,),
                device_id_type=pl.DeviceIdType.MESH,
            )
            r.start()
            amax_rdmas.append(r)
        for r in amax_rdmas:
            r.wait()

        global_amax = jnp.max(amax_ref[:, 0, 0])
        scale = jnp.maximum(global_amax, 1e-30) / 448.0
        for c in range(n // N_CHUNK):
            cs = slice(c * N_CHUNK, (c + 1) * N_CHUNK)
            scaled = jnp.minimum(out_ref[:, cs] / scale, 448.0)
            q = scaled.astype(jnp.float8_e4m3fn)
            out_ref[:, cs] = q.astype(jnp.float32) * scale

    return pl.pallas_call(
        body,
        out_shape=jax.ShapeDtypeStruct((m_blk, n), jnp.float32),
        in_specs=[
            pl.BlockSpec(memory_space=pltpu.MemorySpace.HBM),
            pl.BlockSpec(memory_space=pltpu.VMEM),
        ],
        out_specs=pl.BlockSpec(memory_space=pltpu.VMEM),
        scratch_shapes=[
            pltpu.VMEM((2, m_blk, k_loc), jnp.float32),
            pltpu.VMEM((3, m_blk, k_loc), jnp.bfloat16),
            pltpu.VMEM((2, CH, m_blk, N_CHUNK), jnp.float32),
            pltpu.VMEM((2, CH, m_blk, N_CHUNK), jnp.float32),
            pltpu.VMEM((N_DEV, 8, 128), jnp.float32),
            pltpu.SemaphoreType.DMA((2,)),
            pltpu.SemaphoreType.DMA(((N_DEV - 1) * CH,)),
            pltpu.SemaphoreType.DMA(((N_DEV - 1) * CH,)),
            pltpu.SemaphoreType.DMA(((N_DEV - 1) * CH,)),
            pltpu.SemaphoreType.DMA(((N_DEV - 1) * CH,)),
            pltpu.SemaphoreType.DMA((N_DEV - 1,)),
            pltpu.SemaphoreType.DMA((N_DEV - 1,)),
            pltpu.SemaphoreType.REGULAR,
            pltpu.SemaphoreType.REGULAR,
        ],
        compiler_params=pltpu.CompilerParams(
            collective_id=0,
            vmem_limit_bytes=50 * 1024 * 1024,
        ),
    )(x, w_mat)


# device time: 172434 ns/iter; 1.8402x vs baseline; 1.0003x over previous
import jax
import jax.numpy as jnp
from jax import lax
from jax.experimental import pallas as pl
from jax.experimental.pallas import tpu as pltpu

N_DEV = 4
N_CHUNK = 512
CH = 2


def kernel(x, w_mat):
    m_glob, k_loc = x.shape
    k2, n = w_mat.shape
    assert k_loc == k2
    m_blk = m_glob // N_DEV
    n_half = n // 2
    assert n_half == CH * N_CHUNK

    w_mat = w_mat.astype(jnp.bfloat16)

    def body(x_hbm, wb_ref, out_ref, stage_ref, xb_ref,
             cw_ref, ccw_ref, amax_ref,
             x_sems, cw_send_sems, cw_recv_sems, ccw_send_sems, ccw_recv_sems,
             amax_send_sems, amax_recv_sems, credit_cw, credit_ccw):
        my = lax.axis_index("i")
        left = lax.rem(my + (N_DEV - 1), N_DEV)
        right = lax.rem(my + 1, N_DEV)

        barrier_sem = pltpu.get_barrier_semaphore()
        for nbr in (left, right):
            pl.semaphore_signal(
                barrier_sem, inc=1,
                device_id=(nbr,), device_id_type=pl.DeviceIdType.MESH,
            )
        pl.semaphore_wait(barrier_sem, 2)

        def fetch_x_block(block_idx, slot):
            return pltpu.make_async_copy(
                x_hbm.at[pl.ds(block_idx * m_blk, m_blk), :],
                stage_ref.at[slot],
                x_sems.at[slot],
            )

        def cast_block(stage_slot, xb_slot):
            for r in range(4):
                rs = slice(r * (m_blk // 4), (r + 1) * (m_blk // 4))
                xb_ref[xb_slot, rs] = stage_ref[stage_slot, rs].astype(
                    jnp.bfloat16)

        def chunk_dot(slot, w_col0, c):
            ws_ = slice(w_col0 + c * N_CHUNK, w_col0 + (c + 1) * N_CHUNK)
            return lax.dot_general(
                xb_ref[slot], wb_ref[:, ws_], (((1,), (0,)), ((), ())),
                preferred_element_type=jnp.float32,
            )

        send_slot = (0, 1, 0)
        recv_slot = (1, 0, None)

        def make_hop(lane, s, c):
            ref = cw_ref if lane == 0 else ccw_ref
            ssem = cw_send_sems if lane == 0 else ccw_send_sems
            rsem = cw_recv_sems if lane == 0 else ccw_recv_sems
            tgt = right if lane == 0 else left
            base = 0 if lane == 0 else n_half
            last = s == N_DEV - 2
            return pltpu.make_async_remote_copy(
                src_ref=ref.at[send_slot[s], c],
                dst_ref=(out_ref.at[:, pl.ds(base + c * N_CHUNK, N_CHUNK)]
                         if last else ref.at[recv_slot[s], c]),
                send_sem=ssem.at[CH * s + c],
                recv_sem=rsem.at[CH * s + c],
                device_id=(tgt,),
                device_id_type=pl.DeviceIdType.MESH,
            )

        dma_a = fetch_x_block(lax.rem(my + 3, N_DEV), 0)
        dma_b = fetch_x_block(lax.rem(my + 1, N_DEV), 1)
        dma_a.start()
        dma_b.start()

        rdmas = {}
        dma_a.wait()
        cast_block(0, 0)
        for c in range(CH):
            cw_ref[0, c] = chunk_dot(0, 0, c)
            r = make_hop(0, 0, c)
            r.start()
            rdmas[(0, 0, c)] = r
        dma_c = fetch_x_block(lax.rem(my + 2, N_DEV), 0)
        dma_c.start()
        dma_b.wait()
        cast_block(1, 1)
        for c in range(CH):
            ccw_ref[0, c] = chunk_dot(1, n_half, c)
            r = make_hop(1, 0, c)
            r.start()
            rdmas[(1, 0, c)] = r
        dma_my = fetch_x_block(my, 1)
        dma_my.start()

        cw_stage = (2, 1, 2)
        ccw_stage = (2, 0, 2)

        local_amax = jnp.float32(0.0)
        for s in range(N_DEV - 1):
            last = s == N_DEV - 2
            if s == 0:
                dma_c.wait()
                cast_block(0, 2)
            if last:
                dma_my.wait()
                cast_block(1, 2)
            for c in range(CH):
                for lane in range(2):
                    ref = cw_ref if lane == 0 else ccw_ref
                    st = (cw_stage if lane == 0 else ccw_stage)[s]
                    base = 0 if lane == 0 else n_half
                    cs = slice(base + c * N_CHUNK, base + (c + 1) * N_CHUNK)
                    p = chunk_dot(st, base, c)
                    rdmas[(lane, s, c)].wait()
                    if last:
                        v = jnp.maximum(out_ref[:, cs] + p, 0.0)
                        out_ref[:, cs] = v
                        local_amax = jnp.maximum(local_amax, jnp.max(v))
                    else:
                        ref[recv_slot[s], c] = ref[recv_slot[s], c] + p
                        if s == 1:
                            nr = make_hop(lane, 2, c)
                            nr.start()
                            rdmas[(lane, 2, c)] = nr
            if s == 0:
                pl.semaphore_signal(
                    credit_cw, inc=1,
                    device_id=(left,), device_id_type=pl.DeviceIdType.MESH,
                )
                pl.semaphore_signal(
                    credit_ccw, inc=1,
                    device_id=(right,), device_id_type=pl.DeviceIdType.MESH,
                )
                pl.semaphore_wait(credit_cw, 1)
                pl.semaphore_wait(credit_ccw, 1)
                for c in range(CH):
                    for lane in range(2):
                        nr = make_hop(lane, 1, c)
                        nr.start()
                        rdmas[(lane, 1, c)] = nr

        amax_ref[0] = jnp.full((8, 128), local_amax, jnp.float32)
        amax_rdmas = []
        for k in range(1, N_DEV):
            tgt = lax.rem(my + k, N_DEV)
            r = pltpu.make_async_remote_copy(
                src_ref=amax_ref.at[0],
                dst_ref=amax_ref.at[k],
                send_sem=amax_send_sems.at[k - 1],
                recv_sem=amax_recv_sems.at[k - 1],
                device_id=(tgt,),
                device_id_type=pl.DeviceIdType.MESH,
            )
            r.start()
            amax_rdmas.append(r)
        for r in amax_rdmas:
            r.wait()

        global_amax = jnp.max(amax_ref[:, 0, 0])
        scale = jnp.maximum(global_amax, 1e-30) / 448.0
        inv_scale = 1.0 / scale
        for c in range(n // N_CHUNK):
            cs = slice(c * N_CHUNK, (c + 1) * N_CHUNK)
            scaled = jnp.minimum(out_ref[:, cs] * inv_scale, 448.0)
            q = scaled.astype(jnp.float8_e4m3fn)
            out_ref[:, cs] = q.astype(jnp.float32) * scale

    return pl.pallas_call(
        body,
        out_shape=jax.ShapeDtypeStruct((m_blk, n), jnp.float32),
        in_specs=[
            pl.BlockSpec(memory_space=pltpu.MemorySpace.HBM),
            pl.BlockSpec(memory_space=pltpu.VMEM),
        ],
        out_specs=pl.BlockSpec(memory_space=pltpu.VMEM),
        scratch_shapes=[
            pltpu.VMEM((2, m_blk, k_loc), jnp.float32),
            pltpu.VMEM((3, m_blk, k_loc), jnp.bfloat16),
            pltpu.VMEM((2, CH, m_blk, N_CHUNK), jnp.float32),
            pltpu.VMEM((2, CH, m_blk, N_CHUNK), jnp.float32),
            pltpu.VMEM((N_DEV, 8, 128), jnp.float32),
            pltpu.SemaphoreType.DMA((2,)),
            pltpu.SemaphoreType.DMA(((N_DEV - 1) * CH,)),
            pltpu.SemaphoreType.DMA(((N_DEV - 1) * CH,)),
            pltpu.SemaphoreType.DMA(((N_DEV - 1) * CH,)),
            pltpu.SemaphoreType.DMA(((N_DEV - 1) * CH,)),
            pltpu.SemaphoreType.DMA((N_DEV - 1,)),
            pltpu.SemaphoreType.DMA((N_DEV - 1,)),
            pltpu.SemaphoreType.REGULAR,
            pltpu.SemaphoreType.REGULAR,
        ],
        compiler_params=pltpu.CompilerParams(
            collective_id=0,
            vmem_limit_bytes=50 * 1024 * 1024,
        ),
    )(x, w_mat)


# device time: 104727 ns/iter; 3.0300x vs baseline; 1.6465x over previous
import jax
import jax.numpy as jnp
from jax import lax
from jax.experimental import pallas as pl
from jax.experimental.pallas import tpu as pltpu

N_DEV = 4
N_CHUNK = 512
CH = 2


def kernel(x, w_mat):
    m_glob, k_loc = x.shape
    k2, n = w_mat.shape
    assert k_loc == k2
    m_blk = m_glob // N_DEV
    n_half = n // 2
    assert n_half == CH * N_CHUNK

    w_mat = w_mat.astype(jnp.bfloat16)

    def body(x_hbm, wb_ref, out_ref, stage_ref, xb_ref,
             cw_ref, ccw_ref, fin_ref, amax_ref,
             x_sems, cw_send_sems, cw_recv_sems, ccw_send_sems, ccw_recv_sems,
             amax_send_sems, amax_recv_sems, credit_cw, credit_ccw):
        my = lax.axis_index("i")
        left = lax.rem(my + (N_DEV - 1), N_DEV)
        right = lax.rem(my + 1, N_DEV)

        barrier_sem = pltpu.get_barrier_semaphore()
        for nbr in (left, right):
            pl.semaphore_signal(
                barrier_sem, inc=1,
                device_id=(nbr,), device_id_type=pl.DeviceIdType.MESH,
            )
        pl.semaphore_wait(barrier_sem, 2)

        def fetch_x_block(block_idx, slot):
            return pltpu.make_async_copy(
                x_hbm.at[pl.ds(block_idx * m_blk, m_blk), :],
                stage_ref.at[slot],
                x_sems.at[slot],
            )

        def cast_block(stage_slot, xb_slot):
            for r in range(4):
                rs = slice(r * (m_blk // 4), (r + 1) * (m_blk // 4))
                xb_ref[xb_slot, rs] = stage_ref[stage_slot, rs].astype(
                    jnp.bfloat16)

        def chunk_dot(slot, w_col0, c):
            ws_ = slice(w_col0 + c * N_CHUNK, w_col0 + (c + 1) * N_CHUNK)
            return lax.dot_general(
                xb_ref[slot], wb_ref[:, ws_], (((1,), (0,)), ((), ())),
                preferred_element_type=jnp.float32,
            )

        send_slot = (0, 1, 0)
        recv_slot = (1, 0, None)

        def make_hop(lane, s, c):
            ref = cw_ref if lane == 0 else ccw_ref
            ssem = cw_send_sems if lane == 0 else ccw_send_sems
            rsem = cw_recv_sems if lane == 0 else ccw_recv_sems
            tgt = right if lane == 0 else left
            last = s == N_DEV - 2
            return pltpu.make_async_remote_copy(
                src_ref=ref.at[send_slot[s], c],
                dst_ref=(fin_ref.at[lane, c] if last
                         else ref.at[recv_slot[s], c]),
                send_sem=ssem.at[CH * s + c],
                recv_sem=rsem.at[CH * s + c],
                device_id=(tgt,),
                device_id_type=pl.DeviceIdType.MESH,
            )

        dma_a = fetch_x_block(lax.rem(my + 3, N_DEV), 0)
        dma_b = fetch_x_block(lax.rem(my + 1, N_DEV), 1)
        dma_a.start()
        dma_b.start()

        rdmas = {}
        dma_a.wait()
        cast_block(0, 0)
        for c in range(CH):
            cw_ref[0, c] = chunk_dot(0, 0, c).astype(jnp.bfloat16)
            r = make_hop(0, 0, c)
            r.start()
            rdmas[(0, 0, c)] = r
        dma_c = fetch_x_block(lax.rem(my + 2, N_DEV), 0)
        dma_c.start()
        dma_b.wait()
        cast_block(1, 1)
        for c in range(CH):
            ccw_ref[0, c] = chunk_dot(1, n_half, c).astype(jnp.bfloat16)
            r = make_hop(1, 0, c)
            r.start()
            rdmas[(1, 0, c)] = r
        dma_my = fetch_x_block(my, 1)
        dma_my.start()

        cw_stage = (2, 1, 2)
        ccw_stage = (2, 0, 2)

        local_amax = jnp.float32(0.0)
        for s in range(N_DEV - 1):
            last = s == N_DEV - 2
            if s == 0:
                dma_c.wait()
                cast_block(0, 2)
            if last:
                dma_my.wait()
                cast_block(1, 2)
            for c in range(CH):
                for lane in range(2):
                    ref = cw_ref if lane == 0 else ccw_ref
                    st = (cw_stage if lane == 0 else ccw_stage)[s]
                    base = 0 if lane == 0 else n_half
                    cs = slice(base + c * N_CHUNK, base + (c + 1) * N_CHUNK)
                    p = chunk_dot(st, base, c)
                    rdmas[(lane, s, c)].wait()
                    if last:
                        v = jnp.maximum(
                            fin_ref[lane, c].astype(jnp.float32) + p, 0.0)
                        out_ref[:, cs] = v
                        local_amax = jnp.maximum(local_amax, jnp.max(v))
                    else:
                        acc = ref[recv_slot[s], c].astype(jnp.float32) + p
                        ref[recv_slot[s], c] = acc.astype(jnp.bfloat16)
                        if s == 1:
                            nr = make_hop(lane, 2, c)
                            nr.start()
                            rdmas[(lane, 2, c)] = nr
            if s == 0:
                pl.semaphore_signal(
                    credit_cw, inc=1,
                    device_id=(left,), device_id_type=pl.DeviceIdType.MESH,
                )
                pl.semaphore_signal(
                    credit_ccw, inc=1,
                    device_id=(right,), device_id_type=pl.DeviceIdType.MESH,
                )
                pl.semaphore_wait(credit_cw, 1)
                pl.semaphore_wait(credit_ccw, 1)
                for c in range(CH):
                    for lane in range(2):
                        nr = make_hop(lane, 1, c)
                        nr.start()
                        rdmas[(lane, 1, c)] = nr

        amax_ref[0] = jnp.full((8, 128), local_amax, jnp.float32)
        amax_rdmas = []
        for k in range(1, N_DEV):
            tgt = lax.rem(my + k, N_DEV)
            r = pltpu.make_async_remote_copy(
                src_ref=amax_ref.at[0],
                dst_ref=amax_ref.at[k],
                send_sem=amax_send_sems.at[k - 1],
                recv_sem=amax_recv_sems.at[k - 1],
                device_id=(tgt,),
                device_id_type=pl.DeviceIdType.MESH,
            )
            r.start()
            amax_rdmas.append(r)
        for r in amax_rdmas:
            r.wait()

        global_amax = jnp.max(amax_ref[:, 0, 0])
        scale = jnp.maximum(global_amax, 1e-30) / 448.0
        inv_scale = 1.0 / scale
        for c in range(n // N_CHUNK):
            cs = slice(c * N_CHUNK, (c + 1) * N_CHUNK)
            scaled = jnp.minimum(out_ref[:, cs] * inv_scale, 448.0)
            q = scaled.astype(jnp.float8_e4m3fn)
            out_ref[:, cs] = q.astype(jnp.float32) * scale

    return pl.pallas_call(
        body,
        out_shape=jax.ShapeDtypeStruct((m_blk, n), jnp.float32),
        in_specs=[
            pl.BlockSpec(memory_space=pltpu.MemorySpace.HBM),
            pl.BlockSpec(memory_space=pltpu.VMEM),
        ],
        out_specs=pl.BlockSpec(memory_space=pltpu.VMEM),
        scratch_shapes=[
            pltpu.VMEM((2, m_blk, k_loc), jnp.float32),
            pltpu.VMEM((3, m_blk, k_loc), jnp.bfloat16),
            pltpu.VMEM((2, CH, m_blk, N_CHUNK), jnp.bfloat16),
            pltpu.VMEM((2, CH, m_blk, N_CHUNK), jnp.bfloat16),
            pltpu.VMEM((2, CH, m_blk, N_CHUNK), jnp.bfloat16),
            pltpu.VMEM((N_DEV, 8, 128), jnp.float32),
            pltpu.SemaphoreType.DMA((2,)),
            pltpu.SemaphoreType.DMA(((N_DEV - 1) * CH,)),
            pltpu.SemaphoreType.DMA(((N_DEV - 1) * CH,)),
            pltpu.SemaphoreType.DMA(((N_DEV - 1) * CH,)),
            pltpu.SemaphoreType.DMA(((N_DEV - 1) * CH,)),
            pltpu.SemaphoreType.DMA((N_DEV - 1,)),
            pltpu.SemaphoreType.DMA((N_DEV - 1,)),
            pltpu.SemaphoreType.REGULAR,
            pltpu.SemaphoreType.REGULAR,
        ],
        compiler_params=pltpu.CompilerParams(
            collective_id=0,
            vmem_limit_bytes=50 * 1024 * 1024,
        ),
    )(x, w_mat)
